# Initial kernel scaffold; baseline (speedup 1.0000x reference)
import jax
import jax.numpy as jnp
from jax.experimental import pallas as pl


def kernel(x, rnn_hxs, batch_mask, W_ih, W_hh, b_ih, b_hh):
    raise NotImplementedError("write your pallas kernel here")


# speedup vs baseline: 3.1499x; 3.1499x over previous
import functools

import jax
import jax.numpy as jnp
from jax import lax
from jax.experimental import pallas as pl
from jax.experimental.pallas import tpu as pltpu
from jax.experimental.pallas import tpu_sc as plsc

_T, _B, _D, _H = 512, 16, 512, 512
_TB = _T * _B
_NW = 32
_RPW = _TB // _NW
_CH = 64


def _sc_mesh():
    return plsc.VectorSubcoreMesh(core_axis_name="c", subcore_axis_name="s")


def _cumsum16(v, tmp_v, iota):
    for s in (1, 2, 4, 8):
        tmp_v[...] = v
        sh = plsc.load_gather(tmp_v, [jnp.maximum(iota - s, 0)])
        v = v + jnp.where(iota >= s, sh, 0)
    return v


def _build_index_kernel():
    mesh = _sc_mesh()

    @functools.partial(
        pl.kernel,
        mesh=mesh,
        compiler_params=pltpu.CompilerParams(needs_layout_passes=False),
        out_type=[
            jax.ShapeDtypeStruct((_TB,), jnp.int32),
            jax.ShapeDtypeStruct((_TB,), jnp.int32),
            jax.ShapeDtypeStruct((_B,), jnp.int32),
        ],
        scratch_types=[
            pltpu.VMEM((_TB,), jnp.int32),
            pltpu.VMEM((_TB + _B,), jnp.int32),
            pltpu.VMEM((_TB,), jnp.int32),
            pltpu.VMEM((_TB,), jnp.int32),
            pltpu.VMEM((_B,), jnp.int32),
            pltpu.VMEM((_B,), jnp.int32),
        ],
    )
    def idx_kernel(bm_hbm, gidx_hbm, gidx2_hbm, len_hbm,
                   bm_v, perm_v, gidx_v, gidx2_v, len_v, tmp_v):
        wid = lax.axis_index("s") * 2 + lax.axis_index("c")

        @pl.when(wid == 0)
        def _():
            pltpu.sync_copy(bm_hbm, bm_v)
            iota = lax.iota(jnp.int32, _B)
            zeros = jnp.zeros((_B,), jnp.int32)

            def p1(t, carry):
                k0, lacc = carry
                bm = bm_v[pl.ds(t * _B, _B)]
                on = bm > 0
                cs = _cumsum16(bm, tmp_v, iota)
                rank = cs - bm + k0
                plsc.store_scatter(perm_v, [jnp.where(on, rank, _TB + iota)],
                                   iota + t * _B)
                gidx2_v[pl.ds(t * _B, _B)] = jnp.where(on, rank, _TB)
                return (k0 + plsc.all_reduce_population_count(on), lacc + bm)

            total, lvec = lax.fori_loop(0, _T, p1, (zeros, zeros))
            len_v[...] = lvec

            def p2(t, k0):
                on = lvec > t
                pmi = jnp.where(on, 1, 0)
                cs = _cumsum16(pmi, tmp_v, iota)
                rank = jnp.minimum(cs - pmi + k0, _TB - 1)
                g = plsc.load_gather(perm_v, [rank])
                gidx_v[pl.ds(t * _B, _B)] = jnp.where(on, g, 0)
                return k0 + plsc.all_reduce_population_count(on)

            lax.fori_loop(0, _T, p2, zeros)

            pltpu.sync_copy(gidx_v, gidx_hbm)
            pltpu.sync_copy(gidx2_v, gidx2_hbm)
            pltpu.sync_copy(len_v, len_hbm)

    return idx_kernel


def _build_gather_kernel():
    mesh = _sc_mesh()

    @functools.partial(
        pl.kernel,
        mesh=mesh,
        compiler_params=pltpu.CompilerParams(needs_layout_passes=False),
        out_type=jax.ShapeDtypeStruct((_TB, _H), jnp.float32),
        scratch_types=[
            pltpu.VMEM((_CH,), jnp.int32),
            pltpu.VMEM((_CH, _H), jnp.float32),
            pltpu.SemaphoreType.DMA,
        ],
    )
    def gather_kernel(table_hbm, idx_hbm, out_hbm, idx_v, rows_v, sem):
        wid = lax.axis_index("s") * 2 + lax.axis_index("c")
        base_w = wid * _RPW

        def body(c, _):
            base = base_w + c * _CH
            pltpu.sync_copy(idx_hbm.at[pl.ds(base, _CH)], idx_v)
            pltpu.async_copy(table_hbm.at[idx_v], rows_v, sem).wait()
            pltpu.sync_copy(rows_v, out_hbm.at[pl.ds(base, _CH)])
            return 0

        lax.fori_loop(0, _RPW // _CH, body, 0)

    return gather_kernel


_TM = 512


def _gi_gemm_body(len_ref, a_ref, w_ref, b_ref, o_ref):
    i = pl.program_id(0)
    maxl = jnp.max(len_ref[...])

    @pl.when(i * _TM < maxl * _B)
    def _():
        acc = lax.dot_general(a_ref[...], w_ref[...],
                              (((1,), (1,)), ((), ())),
                              preferred_element_type=jnp.float32)
        o_ref[...] = acc + b_ref[...]


def _gi_gemm(lcol, compact, w_ih, b_ih_row):
    return pl.pallas_call(
        _gi_gemm_body,
        grid=(_TB // _TM,),
        in_specs=[
            pl.BlockSpec((_B, 1), lambda i: (0, 0)),
            pl.BlockSpec((_TM, _D), lambda i: (i, 0)),
            pl.BlockSpec((3 * _H, _D), lambda i: (0, 0)),
            pl.BlockSpec((1, 3 * _H), lambda i: (0, 0)),
        ],
        out_specs=pl.BlockSpec((_TM, 3 * _H), lambda i: (i, 0)),
        out_shape=jax.ShapeDtypeStruct((_TB, 3 * _H), jnp.float32),
    )(lcol, compact, w_ih, b_ih_row)


def _scan_body(len_ref, gi_ref, h0_ref, w_ref, bhh_ref, y_ref, hout_ref, h_v):
    t = pl.program_id(0)
    lvec = len_ref[...]
    maxl = jnp.max(lvec)

    @pl.when(t == 0)
    def _():
        h_v[...] = h0_ref[...]

    @pl.when(jnp.logical_and(t < maxl, t < _T))
    def _():
        h = h_v[...]
        gi = gi_ref[0]
        gh = lax.dot_general(h, w_ref[...], (((1,), (1,)), ((), ())),
                             preferred_element_type=jnp.float32) + bhh_ref[...]
        r = jax.nn.sigmoid(gi[:, :_H] + gh[:, :_H])
        z = jax.nn.sigmoid(gi[:, _H:2 * _H] + gh[:, _H:2 * _H])
        n = jnp.tanh(gi[:, 2 * _H:] + r * gh[:, 2 * _H:])
        hn = (1.0 - z) * n + z * h
        valid = lvec > t
        h_v[...] = jnp.where(valid, hn, h)
        y_ref[0] = jnp.where(valid, hn, 0.0)

    @pl.when(t == _T)
    def _():
        y_ref[0] = jnp.zeros((_B, _H), jnp.float32)
        hout_ref[...] = h_v[...]


def _gru_scan(lcol, gi3, h0, w_hh, b_hh_row):
    return pl.pallas_call(
        _scan_body,
        grid=(_T + 1,),
        in_specs=[
            pl.BlockSpec((_B, 1), lambda t: (0, 0)),
            pl.BlockSpec((1, _B, 3 * _H), lambda t: (jnp.minimum(t, _T - 1), 0, 0)),
            pl.BlockSpec((_B, _H), lambda t: (0, 0)),
            pl.BlockSpec((3 * _H, _H), lambda t: (0, 0)),
            pl.BlockSpec((1, 3 * _H), lambda t: (0, 0)),
        ],
        out_specs=[
            pl.BlockSpec((1, _B, _H), lambda t: (t, 0, 0)),
            pl.BlockSpec((_B, _H), lambda t: (0, 0)),
        ],
        out_shape=[
            jax.ShapeDtypeStruct((_T + 1, _B, _H), jnp.float32),
            jax.ShapeDtypeStruct((_B, _H), jnp.float32),
        ],
        scratch_shapes=[pltpu.VMEM((_B, _H), jnp.float32)],
    )(lcol, gi3, h0, w_hh, b_hh_row)


_idx_call = _build_index_kernel()
_gather_rows = _build_gather_kernel()


def kernel(x, rnn_hxs, batch_mask, W_ih, W_hh, b_ih, b_hh):
    x2d = x.reshape(_TB, _D)
    bm = batch_mask.reshape(_TB).astype(jnp.int32)

    gidx, gidx2, lvec = _idx_call(bm)
    lcol = lvec.reshape(_B, 1)

    compact = _gather_rows(x2d, gidx)
    gi = _gi_gemm(lcol, compact, W_ih, b_ih.reshape(1, 3 * _H))
    ypad, h_fin = _gru_scan(lcol, gi.reshape(_T, _B, 3 * _H), rnn_hxs[0],
                            W_hh, b_hh.reshape(1, 3 * _H))
    ypad2d = ypad.reshape((_T + 1) * _B, _H)
    scores2d = _gather_rows(ypad2d, gidx2)
    return scores2d.reshape(_T, _B, _H), h_fin[None]


# speedup vs baseline: 3.1609x; 1.0035x over previous
import functools

import jax
import jax.numpy as jnp
from jax import lax
from jax.experimental import pallas as pl
from jax.experimental.pallas import tpu as pltpu
from jax.experimental.pallas import tpu_sc as plsc

_T, _B, _D, _H = 512, 16, 512, 512
_TB = _T * _B
_NW = 32
_RPW = _TB // _NW
_CH = 64


def _sc_mesh():
    return plsc.VectorSubcoreMesh(core_axis_name="c", subcore_axis_name="s")


def _cumsum16(v, tmp_v, iota):
    for s in (1, 2, 4, 8):
        tmp_v[...] = v
        sh = plsc.load_gather(tmp_v, [jnp.maximum(iota - s, 0)])
        v = v + jnp.where(iota >= s, sh, 0)
    return v


def _build_index_kernel():
    mesh = _sc_mesh()

    @functools.partial(
        pl.kernel,
        mesh=mesh,
        compiler_params=pltpu.CompilerParams(needs_layout_passes=False),
        out_type=[
            jax.ShapeDtypeStruct((_TB,), jnp.int32),
            jax.ShapeDtypeStruct((_TB,), jnp.int32),
            jax.ShapeDtypeStruct((_B,), jnp.int32),
        ],
        scratch_types=[
            pltpu.VMEM((_TB,), jnp.int32),
            pltpu.VMEM((_TB + _B,), jnp.int32),
            pltpu.VMEM((_TB,), jnp.int32),
            pltpu.VMEM((_TB,), jnp.int32),
            pltpu.VMEM((_B,), jnp.int32),
            pltpu.VMEM((_B,), jnp.int32),
        ],
    )
    def idx_kernel(bm_hbm, gidx_hbm, gidx2_hbm, len_hbm,
                   bm_v, perm_v, gidx_v, gidx2_v, len_v, tmp_v):
        wid = lax.axis_index("s") * 2 + lax.axis_index("c")

        @pl.when(wid == 0)
        def _():
            pltpu.sync_copy(bm_hbm, bm_v)
            iota = lax.iota(jnp.int32, _B)
            zeros = jnp.zeros((_B,), jnp.int32)

            def p1(t, carry):
                k0, lacc = carry
                bm = bm_v[pl.ds(t * _B, _B)]
                on = bm > 0
                cs = _cumsum16(bm, tmp_v, iota)
                rank = cs - bm + k0
                plsc.store_scatter(perm_v, [jnp.where(on, rank, _TB + iota)],
                                   iota + t * _B)
                gidx2_v[pl.ds(t * _B, _B)] = jnp.where(on, rank, _TB)
                return (k0 + plsc.all_reduce_population_count(on), lacc + bm)

            total, lvec = lax.fori_loop(0, _T, p1, (zeros, zeros))
            len_v[...] = lvec

            def p2(t, k0):
                on = lvec > t
                pmi = jnp.where(on, 1, 0)
                cs = _cumsum16(pmi, tmp_v, iota)
                rank = jnp.minimum(cs - pmi + k0, _TB - 1)
                g = plsc.load_gather(perm_v, [rank])
                gidx_v[pl.ds(t * _B, _B)] = jnp.where(on, g, 0)
                return k0 + plsc.all_reduce_population_count(on)

            lax.fori_loop(0, _T, p2, zeros)

            pltpu.sync_copy(gidx_v, gidx_hbm)
            pltpu.sync_copy(gidx2_v, gidx2_hbm)
            pltpu.sync_copy(len_v, len_hbm)

    return idx_kernel


def _build_gather_kernel():
    mesh = _sc_mesh()
    nch = _RPW // _CH

    @functools.partial(
        pl.kernel,
        mesh=mesh,
        compiler_params=pltpu.CompilerParams(needs_layout_passes=False),
        out_type=jax.ShapeDtypeStruct((_TB, _H), jnp.float32),
        scratch_types=(
            [pltpu.VMEM((_RPW,), jnp.int32)]
            + [pltpu.VMEM((_CH, _H), jnp.float32) for _ in range(3)]
            + [pltpu.SemaphoreType.DMA for _ in range(6)]
        ),
    )
    def gather_kernel(table_hbm, idx_hbm, out_hbm, idx_v, *bufs_sems):
        bufs = bufs_sems[:3]
        gsems = bufs_sems[3:6]
        wsems = bufs_sems[6:9]
        wid = lax.axis_index("s") * 2 + lax.axis_index("c")
        base_w = wid * _RPW
        pltpu.sync_copy(idx_hbm.at[pl.ds(base_w, _RPW)], idx_v)

        def gather(c):
            return pltpu.async_copy(
                table_hbm.at[idx_v.at[pl.ds(c * _CH, _CH)]],
                bufs[c % 3], gsems[c % 3])

        def writeback(c):
            return pltpu.async_copy(
                bufs[c % 3], out_hbm.at[pl.ds(base_w + c * _CH, _CH)],
                wsems[c % 3])

        g = {c: gather(c) for c in range(min(3, nch))}
        w = {}
        for c in range(nch):
            g[c].wait()
            w[c] = writeback(c)
            if c + 3 < nch:
                w[c].wait()
                g[c + 3] = gather(c + 3)
        for c in range(max(0, nch - 3), nch):
            w[c].wait()

    return gather_kernel


_TM = 512


def _gi_gemm_body(len_ref, a_ref, w_ref, b_ref, o_ref):
    i = pl.program_id(0)
    maxl = jnp.max(len_ref[...])

    @pl.when(i * _TM < maxl * _B)
    def _():
        acc = lax.dot_general(a_ref[...], w_ref[...],
                              (((1,), (1,)), ((), ())),
                              preferred_element_type=jnp.float32)
        o_ref[...] = acc + b_ref[...]


def _gi_gemm(lcol, compact, w_ih, b_ih_row):
    return pl.pallas_call(
        _gi_gemm_body,
        grid=(_TB // _TM,),
        in_specs=[
            pl.BlockSpec((_B, 1), lambda i: (0, 0)),
            pl.BlockSpec((_TM, _D), lambda i: (i, 0)),
            pl.BlockSpec((3 * _H, _D), lambda i: (0, 0)),
            pl.BlockSpec((1, 3 * _H), lambda i: (0, 0)),
        ],
        out_specs=pl.BlockSpec((_TM, 3 * _H), lambda i: (i, 0)),
        out_shape=jax.ShapeDtypeStruct((_TB, 3 * _H), jnp.float32),
    )(lcol, compact, w_ih, b_ih_row)


def _scan_body(len_ref, gi_ref, h0_ref, w_ref, bhh_ref, y_ref, hout_ref, h_v):
    t = pl.program_id(0)
    lvec = len_ref[...]
    maxl = jnp.max(lvec)

    @pl.when(t == 0)
    def _():
        h_v[...] = h0_ref[...]

    @pl.when(jnp.logical_and(t < maxl, t < _T))
    def _():
        h = h_v[...]
        gi = gi_ref[0]
        gh = lax.dot_general(h, w_ref[...], (((1,), (1,)), ((), ())),
                             preferred_element_type=jnp.float32) + bhh_ref[...]
        r = jax.nn.sigmoid(gi[:, :_H] + gh[:, :_H])
        z = jax.nn.sigmoid(gi[:, _H:2 * _H] + gh[:, _H:2 * _H])
        n = jnp.tanh(gi[:, 2 * _H:] + r * gh[:, 2 * _H:])
        hn = (1.0 - z) * n + z * h
        valid = lvec > t
        h_v[...] = jnp.where(valid, hn, h)
        y_ref[0] = jnp.where(valid, hn, 0.0)

    @pl.when(t == _T)
    def _():
        y_ref[0] = jnp.zeros((_B, _H), jnp.float32)
        hout_ref[...] = h_v[...]


def _gru_scan(lcol, gi3, h0, w_hh, b_hh_row):
    return pl.pallas_call(
        _scan_body,
        grid=(_T + 1,),
        in_specs=[
            pl.BlockSpec((_B, 1), lambda t: (0, 0)),
            pl.BlockSpec((1, _B, 3 * _H), lambda t: (jnp.minimum(t, _T - 1), 0, 0)),
            pl.BlockSpec((_B, _H), lambda t: (0, 0)),
            pl.BlockSpec((3 * _H, _H), lambda t: (0, 0)),
            pl.BlockSpec((1, 3 * _H), lambda t: (0, 0)),
        ],
        out_specs=[
            pl.BlockSpec((1, _B, _H), lambda t: (t, 0, 0)),
            pl.BlockSpec((_B, _H), lambda t: (0, 0)),
        ],
        out_shape=[
            jax.ShapeDtypeStruct((_T + 1, _B, _H), jnp.float32),
            jax.ShapeDtypeStruct((_B, _H), jnp.float32),
        ],
        scratch_shapes=[pltpu.VMEM((_B, _H), jnp.float32)],
    )(lcol, gi3, h0, w_hh, b_hh_row)


_idx_call = _build_index_kernel()
_gather_rows = _build_gather_kernel()


def kernel(x, rnn_hxs, batch_mask, W_ih, W_hh, b_ih, b_hh):
    x2d = x.reshape(_TB, _D)
    bm = batch_mask.reshape(_TB).astype(jnp.int32)

    gidx, gidx2, lvec = _idx_call(bm)
    lcol = lvec.reshape(_B, 1)

    compact = _gather_rows(x2d, gidx)
    gi = _gi_gemm(lcol, compact, W_ih, b_ih.reshape(1, 3 * _H))
    ypad, h_fin = _gru_scan(lcol, gi.reshape(_T, _B, 3 * _H), rnn_hxs[0],
                            W_hh, b_hh.reshape(1, 3 * _H))
    ypad2d = ypad.reshape((_T + 1) * _B, _H)
    scores2d = _gather_rows(ypad2d, gidx2)
    return scores2d.reshape(_T, _B, _H), h_fin[None]


# speedup vs baseline: 5.2520x; 1.6615x over previous
import functools

import jax
import jax.numpy as jnp
from jax import lax
from jax.experimental import pallas as pl
from jax.experimental.pallas import tpu as pltpu
from jax.experimental.pallas import tpu_sc as plsc

_T, _B, _D, _H = 512, 16, 512, 512
_TB = _T * _B
_NW = 32
_RPW = _TB // _NW
_CH = 64


def _sc_mesh():
    return plsc.VectorSubcoreMesh(core_axis_name="c", subcore_axis_name="s")


def _cumsum16(v, tmp_v, iota):
    for s in (1, 2, 4, 8):
        tmp_v[...] = v
        sh = plsc.load_gather(tmp_v, [jnp.maximum(iota - s, 0)])
        v = v + jnp.where(iota >= s, sh, 0)
    return v


def _build_index_kernel():
    mesh = _sc_mesh()

    @functools.partial(
        pl.kernel,
        mesh=mesh,
        compiler_params=pltpu.CompilerParams(needs_layout_passes=False),
        out_type=[
            jax.ShapeDtypeStruct((_TB,), jnp.int32),
            jax.ShapeDtypeStruct((_TB,), jnp.int32),
            jax.ShapeDtypeStruct((_B,), jnp.int32),
        ],
        scratch_types=[
            pltpu.VMEM((_TB,), jnp.int32),
            pltpu.VMEM((_TB + _B,), jnp.int32),
            pltpu.VMEM((_TB,), jnp.int32),
            pltpu.VMEM((_TB,), jnp.int32),
            pltpu.VMEM((_B,), jnp.int32),
            pltpu.VMEM((_B,), jnp.int32),
        ],
    )
    def idx_kernel(bm_hbm, gidx_hbm, gidx2_hbm, len_hbm,
                   bm_v, perm_v, gidx_v, gidx2_v, len_v, tmp_v):
        wid = lax.axis_index("s") * 2 + lax.axis_index("c")

        @pl.when(wid == 0)
        def _():
            pltpu.sync_copy(bm_hbm, bm_v)
            iota = lax.iota(jnp.int32, _B)
            zeros = jnp.zeros((_B,), jnp.int32)

            def p1(t, carry):
                k0, lacc = carry
                bm = bm_v[pl.ds(t * _B, _B)]
                on = bm > 0
                cs = _cumsum16(bm, tmp_v, iota)
                rank = cs - bm + k0
                plsc.store_scatter(perm_v, [jnp.where(on, rank, _TB + iota)],
                                   iota + t * _B)
                gidx2_v[pl.ds(t * _B, _B)] = jnp.where(on, rank, _TB + iota)
                return (k0 + plsc.all_reduce_population_count(on), lacc + bm)

            total, lvec = lax.fori_loop(0, _T, p1, (zeros, zeros))
            len_v[...] = lvec

            def p2(t, k0):
                on = lvec > t
                pmi = jnp.where(on, 1, 0)
                cs = _cumsum16(pmi, tmp_v, iota)
                rank = jnp.minimum(cs - pmi + k0, _TB - 1)
                g = plsc.load_gather(perm_v, [rank])
                gidx_v[pl.ds(t * _B, _B)] = jnp.where(on, g, iota + t * _B)
                return k0 + plsc.all_reduce_population_count(on)

            lax.fori_loop(0, _T, p2, zeros)

            pltpu.sync_copy(gidx_v, gidx_hbm)
            pltpu.sync_copy(gidx2_v, gidx2_hbm)
            pltpu.sync_copy(len_v, len_hbm)

    return idx_kernel


def _build_gather_kernel():
    mesh = _sc_mesh()
    nch = _RPW // _CH

    @functools.partial(
        pl.kernel,
        mesh=mesh,
        compiler_params=pltpu.CompilerParams(needs_layout_passes=False),
        out_type=jax.ShapeDtypeStruct((_TB, _H), jnp.float32),
        scratch_types=(
            [pltpu.VMEM((_RPW,), jnp.int32)]
            + [pltpu.VMEM((_CH, _H), jnp.float32) for _ in range(3)]
            + [pltpu.SemaphoreType.DMA for _ in range(6)]
        ),
    )
    def gather_kernel(table_hbm, idx_hbm, out_hbm, idx_v, *bufs_sems):
        bufs = bufs_sems[:3]
        gsems = bufs_sems[3:6]
        wsems = bufs_sems[6:9]
        wid = lax.axis_index("s") * 2 + lax.axis_index("c")
        base_w = wid * _RPW
        pltpu.sync_copy(idx_hbm.at[pl.ds(base_w, _RPW)], idx_v)

        def gather(c):
            return pltpu.async_copy(
                table_hbm.at[idx_v.at[pl.ds(c * _CH, _CH)]],
                bufs[c % 3], gsems[c % 3])

        def writeback(c):
            return pltpu.async_copy(
                bufs[c % 3], out_hbm.at[pl.ds(base_w + c * _CH, _CH)],
                wsems[c % 3])

        g = {c: gather(c) for c in range(min(3, nch))}
        w = {}
        for c in range(nch):
            g[c].wait()
            w[c] = writeback(c)
            if c + 3 < nch:
                w[c].wait()
                g[c + 3] = gather(c + 3)
        for c in range(max(0, nch - 3), nch):
            w[c].wait()

    return gather_kernel


_TM = 512


def _gi_gemm_body(len_ref, a_ref, w_ref, b_ref, o_ref):
    i = pl.program_id(0)
    maxl = jnp.max(len_ref[...])

    @pl.when(i * _TM < maxl * _B)
    def _():
        acc = lax.dot_general(a_ref[...], w_ref[...],
                              (((1,), (1,)), ((), ())),
                              preferred_element_type=jnp.float32)
        o_ref[...] = acc + b_ref[...]


def _gi_gemm(lcol, compact, w_ih, b_ih_row):
    return pl.pallas_call(
        _gi_gemm_body,
        grid=(_TB // _TM,),
        in_specs=[
            pl.BlockSpec((_B, 1), lambda i: (0, 0)),
            pl.BlockSpec((_TM, _D), lambda i: (i, 0)),
            pl.BlockSpec((3 * _H, _D), lambda i: (0, 0)),
            pl.BlockSpec((1, 3 * _H), lambda i: (0, 0)),
        ],
        out_specs=pl.BlockSpec((_TM, 3 * _H), lambda i: (i, 0)),
        out_shape=jax.ShapeDtypeStruct((_TB, 3 * _H), jnp.float32),
    )(lcol, compact, w_ih, b_ih_row)


def _scan_body(len_ref, gi_ref, h0_ref, w_ref, bhh_ref, y_ref, hout_ref, h_v):
    t = pl.program_id(0)
    lvec = len_ref[...]
    maxl = jnp.max(lvec)

    @pl.when(t == 0)
    def _():
        h_v[...] = h0_ref[...]

    @pl.when(jnp.logical_and(t < maxl, t < _T))
    def _():
        h = h_v[...]
        gi = gi_ref[0]
        gh = lax.dot_general(h, w_ref[...], (((1,), (1,)), ((), ())),
                             preferred_element_type=jnp.float32) + bhh_ref[...]
        r = jax.nn.sigmoid(gi[:, :_H] + gh[:, :_H])
        z = jax.nn.sigmoid(gi[:, _H:2 * _H] + gh[:, _H:2 * _H])
        n = jnp.tanh(gi[:, 2 * _H:] + r * gh[:, 2 * _H:])
        hn = (1.0 - z) * n + z * h
        valid = lvec > t
        h_v[...] = jnp.where(valid, hn, h)
        y_ref[0] = jnp.where(valid, hn, 0.0)

    @pl.when(t == _T)
    def _():
        y_ref[0] = jnp.zeros((_B, _H), jnp.float32)
        hout_ref[...] = h_v[...]


def _gru_scan(lcol, gi3, h0, w_hh, b_hh_row):
    return pl.pallas_call(
        _scan_body,
        grid=(_T + 1,),
        in_specs=[
            pl.BlockSpec((_B, 1), lambda t: (0, 0)),
            pl.BlockSpec((1, _B, 3 * _H), lambda t: (jnp.minimum(t, _T - 1), 0, 0)),
            pl.BlockSpec((_B, _H), lambda t: (0, 0)),
            pl.BlockSpec((3 * _H, _H), lambda t: (0, 0)),
            pl.BlockSpec((1, 3 * _H), lambda t: (0, 0)),
        ],
        out_specs=[
            pl.BlockSpec((1, _B, _H), lambda t: (t, 0, 0)),
            pl.BlockSpec((_B, _H), lambda t: (0, 0)),
        ],
        out_shape=[
            jax.ShapeDtypeStruct((_T + 1, _B, _H), jnp.float32),
            jax.ShapeDtypeStruct((_B, _H), jnp.float32),
        ],
        scratch_shapes=[pltpu.VMEM((_B, _H), jnp.float32)],
    )(lcol, gi3, h0, w_hh, b_hh_row)


_idx_call = _build_index_kernel()
_gather_rows = _build_gather_kernel()


def kernel(x, rnn_hxs, batch_mask, W_ih, W_hh, b_ih, b_hh):
    x2d = x.reshape(_TB, _D)
    bm = batch_mask.reshape(_TB).astype(jnp.int32)

    gidx, gidx2, lvec = _idx_call(bm)
    lcol = lvec.reshape(_B, 1)

    compact = _gather_rows(x2d, gidx)
    gi = _gi_gemm(lcol, compact, W_ih, b_ih.reshape(1, 3 * _H))
    ypad, h_fin = _gru_scan(lcol, gi.reshape(_T, _B, 3 * _H), rnn_hxs[0],
                            W_hh, b_hh.reshape(1, 3 * _H))
    ypad2d = ypad.reshape((_T + 1) * _B, _H)
    scores2d = _gather_rows(ypad2d, gidx2)
    return scores2d.reshape(_T, _B, _H), h_fin[None]


# speedup vs baseline: 5.3224x; 1.0134x over previous
import functools

import jax
import jax.numpy as jnp
from jax import lax
from jax.experimental import pallas as pl
from jax.experimental.pallas import tpu as pltpu
from jax.experimental.pallas import tpu_sc as plsc

_T, _B, _D, _H = 512, 16, 512, 512
_TB = _T * _B
_NW = 32
_RPW = _TB // _NW
_CH = 64


def _sc_mesh():
    return plsc.VectorSubcoreMesh(core_axis_name="c", subcore_axis_name="s")


def _cumsum16(v, tmp_v, iota):
    for s in (1, 2, 4, 8):
        tmp_v[...] = v
        sh = plsc.load_gather(tmp_v, [jnp.maximum(iota - s, 0)])
        v = v + jnp.where(iota >= s, sh, 0)
    return v


def _build_index_kernel():
    mesh = _sc_mesh()

    @functools.partial(
        pl.kernel,
        mesh=mesh,
        compiler_params=pltpu.CompilerParams(needs_layout_passes=False),
        out_type=[
            jax.ShapeDtypeStruct((_TB,), jnp.int32),
            jax.ShapeDtypeStruct((_TB,), jnp.int32),
            jax.ShapeDtypeStruct((_B,), jnp.int32),
        ],
        scratch_types=[
            pltpu.VMEM((_TB,), jnp.int32),
            pltpu.VMEM((_TB + _B,), jnp.int32),
            pltpu.VMEM((_TB,), jnp.int32),
            pltpu.VMEM((_TB,), jnp.int32),
            pltpu.VMEM((_B,), jnp.int32),
            pltpu.VMEM((_B,), jnp.int32),
        ],
    )
    def idx_kernel(bm_hbm, gidx_hbm, gidx2_hbm, len_hbm,
                   bm_v, perm_v, gidx_v, gidx2_v, len_v, tmp_v):
        wid = lax.axis_index("s") * 2 + lax.axis_index("c")

        @pl.when(wid == 0)
        def _():
            pltpu.sync_copy(bm_hbm, bm_v)
            iota = lax.iota(jnp.int32, _B)
            zeros = jnp.zeros((_B,), jnp.int32)

            def p1(t, carry):
                k0, lacc = carry
                bm = bm_v[pl.ds(t * _B, _B)]
                on = bm > 0
                cs = _cumsum16(bm, tmp_v, iota)
                rank = cs - bm + k0
                plsc.store_scatter(perm_v, [jnp.where(on, rank, _TB + iota)],
                                   iota + t * _B)
                gidx2_v[pl.ds(t * _B, _B)] = jnp.where(on, rank, _TB + iota)
                return (k0 + plsc.all_reduce_population_count(on), lacc + bm)

            total, lvec = lax.fori_loop(0, _T, p1, (zeros, zeros))
            len_v[...] = lvec

            def p2(t, k0):
                on = lvec > t
                pmi = jnp.where(on, 1, 0)
                cs = _cumsum16(pmi, tmp_v, iota)
                rank = jnp.minimum(cs - pmi + k0, _TB - 1)
                g = plsc.load_gather(perm_v, [rank])
                gidx_v[pl.ds(t * _B, _B)] = jnp.where(on, g, iota + t * _B)
                return k0 + plsc.all_reduce_population_count(on)

            lax.fori_loop(0, _T, p2, zeros)

            pltpu.sync_copy(gidx_v, gidx_hbm)
            pltpu.sync_copy(gidx2_v, gidx2_hbm)
            pltpu.sync_copy(len_v, len_hbm)

    return idx_kernel


def _build_gather_kernel():
    mesh = _sc_mesh()
    nch = _RPW // _CH

    @functools.partial(
        pl.kernel,
        mesh=mesh,
        compiler_params=pltpu.CompilerParams(needs_layout_passes=False),
        out_type=jax.ShapeDtypeStruct((_TB, _H), jnp.float32),
        scratch_types=(
            [pltpu.VMEM((_RPW,), jnp.int32)]
            + [pltpu.VMEM((_CH, _H), jnp.float32) for _ in range(3)]
            + [pltpu.SemaphoreType.DMA for _ in range(6)]
        ),
    )
    def gather_kernel(table_hbm, idx_hbm, out_hbm, idx_v, *bufs_sems):
        bufs = bufs_sems[:3]
        gsems = bufs_sems[3:6]
        wsems = bufs_sems[6:9]
        wid = lax.axis_index("s") * 2 + lax.axis_index("c")
        base_w = wid * _RPW
        pltpu.sync_copy(idx_hbm.at[pl.ds(base_w, _RPW)], idx_v)

        def gather(c):
            return pltpu.async_copy(
                table_hbm.at[idx_v.at[pl.ds(c * _CH, _CH)]],
                bufs[c % 3], gsems[c % 3])

        def writeback(c):
            return pltpu.async_copy(
                bufs[c % 3], out_hbm.at[pl.ds(base_w + c * _CH, _CH)],
                wsems[c % 3])

        g = {c: gather(c) for c in range(min(3, nch))}
        w = {}
        for c in range(nch):
            g[c].wait()
            w[c] = writeback(c)
            if c + 3 < nch:
                w[c].wait()
                g[c + 3] = gather(c + 3)
        for c in range(max(0, nch - 3), nch):
            w[c].wait()

    return gather_kernel


_TM = 512


def _gi_gemm_body(len_ref, a_ref, w_ref, b_ref, o_ref):
    i = pl.program_id(0)
    maxl = jnp.max(len_ref[...])

    @pl.when(i * _TM < maxl * _B)
    def _():
        acc = lax.dot_general(a_ref[...], w_ref[...],
                              (((1,), (1,)), ((), ())),
                              preferred_element_type=jnp.float32)
        o_ref[...] = acc + b_ref[...]


def _gi_gemm(lcol, compact, w_ih, b_ih_row):
    return pl.pallas_call(
        _gi_gemm_body,
        grid=(_TB // _TM,),
        in_specs=[
            pl.BlockSpec((_B, 1), lambda i: (0, 0)),
            pl.BlockSpec((_TM, _D), lambda i: (i, 0)),
            pl.BlockSpec((3 * _H, _D), lambda i: (0, 0)),
            pl.BlockSpec((1, 3 * _H), lambda i: (0, 0)),
        ],
        out_specs=pl.BlockSpec((_TM, 3 * _H), lambda i: (i, 0)),
        out_shape=jax.ShapeDtypeStruct((_TB, 3 * _H), jnp.float32),
    )(lcol, compact, w_ih, b_ih_row)


def _scan_body(len_ref, gi_ref, h0_ref, w_ref, bhh_ref, y_ref, hout_ref, h_v):
    t = pl.program_id(0)
    lvec = len_ref[...]
    maxl = jnp.max(lvec)

    @pl.when(t == 0)
    def _():
        h_v[...] = h0_ref[...]

    @pl.when(jnp.logical_and(t < maxl, t < _T))
    def _():
        h = h_v[...]
        gi = gi_ref[0]
        gh = lax.dot_general(h.astype(jnp.bfloat16), w_ref[...],
                             (((1,), (1,)), ((), ())),
                             preferred_element_type=jnp.float32) + bhh_ref[...]
        r = jax.nn.sigmoid(gi[:, :_H] + gh[:, :_H])
        z = jax.nn.sigmoid(gi[:, _H:2 * _H] + gh[:, _H:2 * _H])
        n = jnp.tanh(gi[:, 2 * _H:] + r * gh[:, 2 * _H:])
        hn = (1.0 - z) * n + z * h
        valid = lvec > t
        h_v[...] = jnp.where(valid, hn, h)
        y_ref[0] = jnp.where(valid, hn, 0.0)

    @pl.when(t == _T)
    def _():
        y_ref[0] = jnp.zeros((_B, _H), jnp.float32)
        hout_ref[...] = h_v[...]


def _gru_scan(lcol, gi3, h0, w_hh, b_hh_row):
    return pl.pallas_call(
        _scan_body,
        grid=(_T + 1,),
        in_specs=[
            pl.BlockSpec((_B, 1), lambda t: (0, 0)),
            pl.BlockSpec((1, _B, 3 * _H), lambda t: (jnp.minimum(t, _T - 1), 0, 0)),
            pl.BlockSpec((_B, _H), lambda t: (0, 0)),
            pl.BlockSpec((3 * _H, _H), lambda t: (0, 0)),
            pl.BlockSpec((1, 3 * _H), lambda t: (0, 0)),
        ],
        out_specs=[
            pl.BlockSpec((1, _B, _H), lambda t: (t, 0, 0)),
            pl.BlockSpec((_B, _H), lambda t: (0, 0)),
        ],
        out_shape=[
            jax.ShapeDtypeStruct((_T + 1, _B, _H), jnp.float32),
            jax.ShapeDtypeStruct((_B, _H), jnp.float32),
        ],
        scratch_shapes=[pltpu.VMEM((_B, _H), jnp.float32)],
    )(lcol, gi3, h0, w_hh, b_hh_row)


_idx_call = _build_index_kernel()
_gather_rows = _build_gather_kernel()


def kernel(x, rnn_hxs, batch_mask, W_ih, W_hh, b_ih, b_hh):
    x2d = x.reshape(_TB, _D)
    bm = batch_mask.reshape(_TB).astype(jnp.int32)

    gidx, gidx2, lvec = _idx_call(bm)
    lcol = lvec.reshape(_B, 1)

    compact = _gather_rows(x2d, gidx)
    gi = _gi_gemm(lcol, compact, W_ih, b_ih.reshape(1, 3 * _H))
    ypad, h_fin = _gru_scan(lcol, gi.reshape(_T, _B, 3 * _H), rnn_hxs[0],
                            W_hh.astype(jnp.bfloat16), b_hh.reshape(1, 3 * _H))
    ypad2d = ypad.reshape((_T + 1) * _B, _H)
    scores2d = _gather_rows(ypad2d, gidx2)
    return scores2d.reshape(_T, _B, _H), h_fin[None]


# speedup vs baseline: 8.3238x; 1.5639x over previous
import functools

import jax
import jax.numpy as jnp
from jax import lax
from jax.experimental import pallas as pl
from jax.experimental.pallas import tpu as pltpu
from jax.experimental.pallas import tpu_sc as plsc

_T, _B, _D, _H = 512, 16, 512, 512
_TB = _T * _B
_NW = 32
_RPW = _TB // _NW
_CH = 64


def _sc_mesh():
    return plsc.VectorSubcoreMesh(core_axis_name="c", subcore_axis_name="s")


def _cumsum16(v, tmp_v, iota):
    for s in (1, 2, 4, 8):
        tmp_v[...] = v
        sh = plsc.load_gather(tmp_v, [jnp.maximum(iota - s, 0)])
        v = v + jnp.where(iota >= s, sh, 0)
    return v


def _build_index_kernel():
    mesh = _sc_mesh()

    @functools.partial(
        pl.kernel,
        mesh=mesh,
        compiler_params=pltpu.CompilerParams(needs_layout_passes=False),
        out_type=[
            jax.ShapeDtypeStruct((_TB,), jnp.int32),
            jax.ShapeDtypeStruct((_TB,), jnp.int32),
            jax.ShapeDtypeStruct((_B,), jnp.int32),
        ],
        scratch_types=[
            pltpu.VMEM((_TB,), jnp.int32),
            pltpu.VMEM((_TB + _B,), jnp.int32),
            pltpu.VMEM((_TB,), jnp.int32),
            pltpu.VMEM((_TB,), jnp.int32),
            pltpu.VMEM((_B,), jnp.int32),
            pltpu.VMEM((_B,), jnp.int32),
        ],
    )
    def idx_kernel(bm_hbm, gidx_hbm, gidx2_hbm, len_hbm,
                   bm_v, perm_v, gidx_v, gidx2_v, len_v, tmp_v):
        wid = lax.axis_index("s") * 2 + lax.axis_index("c")

        @pl.when(wid == 0)
        def _():
            pltpu.sync_copy(bm_hbm, bm_v)
            iota = lax.iota(jnp.int32, _B)
            zeros = jnp.zeros((_B,), jnp.int32)

            def p1(t, carry):
                k0, lacc = carry
                bm = bm_v[pl.ds(t * _B, _B)]
                on = bm > 0
                cs = _cumsum16(bm, tmp_v, iota)
                rank = cs - bm + k0
                plsc.store_scatter(perm_v, [jnp.where(on, rank, _TB + iota)],
                                   iota + t * _B)
                gidx2_v[pl.ds(t * _B, _B)] = jnp.where(on, rank, _TB + iota)
                return (k0 + plsc.all_reduce_population_count(on), lacc + bm)

            total, lvec = lax.fori_loop(0, _T, p1, (zeros, zeros))
            len_v[...] = lvec

            def p2(t, k0):
                on = lvec > t
                pmi = jnp.where(on, 1, 0)
                cs = _cumsum16(pmi, tmp_v, iota)
                rank = jnp.minimum(cs - pmi + k0, _TB - 1)
                g = plsc.load_gather(perm_v, [rank])
                gidx_v[pl.ds(t * _B, _B)] = jnp.where(on, g, iota + t * _B)
                return k0 + plsc.all_reduce_population_count(on)

            lax.fori_loop(0, _T, p2, zeros)

            pltpu.sync_copy(gidx_v, gidx_hbm)
            pltpu.sync_copy(gidx2_v, gidx2_hbm)
            pltpu.sync_copy(len_v, len_hbm)

    return idx_kernel


def _build_gather_kernel():
    mesh = _sc_mesh()
    nch = _RPW // _CH

    @functools.partial(
        pl.kernel,
        mesh=mesh,
        compiler_params=pltpu.CompilerParams(needs_layout_passes=False),
        out_type=jax.ShapeDtypeStruct((_TB, _H), jnp.float32),
        scratch_types=(
            [pltpu.VMEM((_RPW,), jnp.int32)]
            + [pltpu.VMEM((_CH, _H), jnp.float32) for _ in range(3)]
            + [pltpu.SemaphoreType.DMA for _ in range(6)]
        ),
    )
    def gather_kernel(table_hbm, idx_hbm, out_hbm, idx_v, *bufs_sems):
        bufs = bufs_sems[:3]
        gsems = bufs_sems[3:6]
        wsems = bufs_sems[6:9]
        wid = lax.axis_index("s") * 2 + lax.axis_index("c")
        base_w = wid * _RPW
        pltpu.sync_copy(idx_hbm.at[pl.ds(base_w, _RPW)], idx_v)

        def gather(c):
            return pltpu.async_copy(
                table_hbm.at[idx_v.at[pl.ds(c * _CH, _CH)]],
                bufs[c % 3], gsems[c % 3])

        def writeback(c):
            return pltpu.async_copy(
                bufs[c % 3], out_hbm.at[pl.ds(base_w + c * _CH, _CH)],
                wsems[c % 3])

        g = {c: gather(c) for c in range(min(3, nch))}
        w = {}
        for c in range(nch):
            g[c].wait()
            w[c] = writeback(c)
            if c + 3 < nch:
                w[c].wait()
                g[c + 3] = gather(c + 3)
        for c in range(max(0, nch - 3), nch):
            w[c].wait()

    return gather_kernel


_TM = 512


def _gi_gemm_body(len_ref, a_ref, w_ref, b_ref, o_ref):
    i = pl.program_id(0)
    maxl = jnp.max(len_ref[...])

    @pl.when(i * _TM < maxl * _B)
    def _():
        acc = lax.dot_general(a_ref[...], w_ref[...],
                              (((1,), (1,)), ((), ())),
                              preferred_element_type=jnp.float32)
        o_ref[...] = acc + b_ref[...]


def _gi_gemm(lcol, compact, w_ih, b_ih_row):
    return pl.pallas_call(
        _gi_gemm_body,
        grid=(_TB // _TM,),
        in_specs=[
            pl.BlockSpec((_B, 1), lambda i: (0, 0)),
            pl.BlockSpec((_TM, _D), lambda i: (i, 0)),
            pl.BlockSpec((3 * _H, _D), lambda i: (0, 0)),
            pl.BlockSpec((1, 3 * _H), lambda i: (0, 0)),
        ],
        out_specs=pl.BlockSpec((_TM, 3 * _H), lambda i: (i, 0)),
        out_shape=jax.ShapeDtypeStruct((_TB, 3 * _H), jnp.float32),
    )(lcol, compact, w_ih, b_ih_row)


_K = 4
_NSC = _T // _K


def _scan_body(len_ref, gi_ref, h0_ref, w_ref, bhh_ref, y_ref, hout_ref, h_v):
    i = pl.program_id(0)
    lvec = len_ref[...]
    maxl = jnp.max(lvec)

    @pl.when(i == 0)
    def _():
        h_v[...] = h0_ref[...]

    @pl.when(jnp.logical_and(i * _K < maxl, i < _NSC))
    def _():
        h = h_v[...]
        for k in range(_K):
            t = i * _K + k
            gi = gi_ref[k]
            gh = lax.dot_general(h.astype(jnp.bfloat16), w_ref[...],
                                 (((1,), (1,)), ((), ())),
                                 preferred_element_type=jnp.float32) + bhh_ref[...]
            r = jax.nn.sigmoid(gi[:, :_H] + gh[:, :_H])
            z = jax.nn.sigmoid(gi[:, _H:2 * _H] + gh[:, _H:2 * _H])
            n = jnp.tanh(gi[:, 2 * _H:] + r * gh[:, 2 * _H:])
            hn = (1.0 - z) * n + z * h
            valid = lvec > t
            h = jnp.where(valid, hn, h)
            y_ref[k] = jnp.where(valid, hn, 0.0)
        h_v[...] = h

    @pl.when(i == _NSC)
    def _():
        y_ref[...] = jnp.zeros((_K, _B, _H), jnp.float32)
        hout_ref[...] = h_v[...]


def _gru_scan(lcol, gi3, h0, w_hh, b_hh_row):
    return pl.pallas_call(
        _scan_body,
        grid=(_NSC + 1,),
        in_specs=[
            pl.BlockSpec((_B, 1), lambda i: (0, 0)),
            pl.BlockSpec((_K, _B, 3 * _H),
                         lambda i: (jnp.minimum(i, _NSC - 1), 0, 0)),
            pl.BlockSpec((_B, _H), lambda i: (0, 0)),
            pl.BlockSpec((3 * _H, _H), lambda i: (0, 0)),
            pl.BlockSpec((1, 3 * _H), lambda i: (0, 0)),
        ],
        out_specs=[
            pl.BlockSpec((_K, _B, _H), lambda i: (i, 0, 0)),
            pl.BlockSpec((_B, _H), lambda i: (0, 0)),
        ],
        out_shape=[
            jax.ShapeDtypeStruct((_T + _K, _B, _H), jnp.float32),
            jax.ShapeDtypeStruct((_B, _H), jnp.float32),
        ],
        scratch_shapes=[pltpu.VMEM((_B, _H), jnp.float32)],
    )(lcol, gi3, h0, w_hh, b_hh_row)


_idx_call = _build_index_kernel()
_gather_rows = _build_gather_kernel()


def kernel(x, rnn_hxs, batch_mask, W_ih, W_hh, b_ih, b_hh):
    x2d = x.reshape(_TB, _D)
    bm = batch_mask.reshape(_TB).astype(jnp.int32)

    gidx, gidx2, lvec = _idx_call(bm)
    lcol = lvec.reshape(_B, 1)

    compact = _gather_rows(x2d, gidx)
    gi = _gi_gemm(lcol, compact, W_ih, b_ih.reshape(1, 3 * _H))
    ypad, h_fin = _gru_scan(lcol, gi.reshape(_T, _B, 3 * _H), rnn_hxs[0],
                            W_hh.astype(jnp.bfloat16), b_hh.reshape(1, 3 * _H))
    ypad2d = ypad.reshape((_T + _K) * _B, _H)
    scores2d = _gather_rows(ypad2d, gidx2)
    return scores2d.reshape(_T, _B, _H), h_fin[None]


# speedup vs baseline: 9.0501x; 1.0873x over previous
import functools

import jax
import jax.numpy as jnp
from jax import lax
from jax.experimental import pallas as pl
from jax.experimental.pallas import tpu as pltpu
from jax.experimental.pallas import tpu_sc as plsc

_T, _B, _D, _H = 512, 16, 512, 512
_TB = _T * _B
_NW = 32
_RPW = _TB // _NW
_CH = 64


def _sc_mesh():
    return plsc.VectorSubcoreMesh(core_axis_name="c", subcore_axis_name="s")


def _cumsum16(v, tmp_v, iota):
    for s in (1, 2, 4, 8):
        tmp_v[...] = v
        sh = plsc.load_gather(tmp_v, [jnp.maximum(iota - s, 0)])
        v = v + jnp.where(iota >= s, sh, 0)
    return v


def _build_index_kernel():
    mesh = _sc_mesh()

    @functools.partial(
        pl.kernel,
        mesh=mesh,
        compiler_params=pltpu.CompilerParams(needs_layout_passes=False),
        out_type=[
            jax.ShapeDtypeStruct((_TB,), jnp.int32),
            jax.ShapeDtypeStruct((_TB,), jnp.int32),
            jax.ShapeDtypeStruct((_B,), jnp.int32),
        ],
        scratch_types=[
            pltpu.VMEM((_TB,), jnp.int32),
            pltpu.VMEM((_TB + _B,), jnp.int32),
            pltpu.VMEM((_TB,), jnp.int32),
            pltpu.VMEM((_TB,), jnp.int32),
            pltpu.VMEM((_B,), jnp.int32),
            pltpu.VMEM((_B,), jnp.int32),
        ],
    )
    def idx_kernel(bm_hbm, gidx_hbm, gidx2_hbm, len_hbm,
                   bm_v, perm_v, gidx_v, gidx2_v, len_v, tmp_v):
        wid = lax.axis_index("s") * 2 + lax.axis_index("c")

        @pl.when(wid == 0)
        def _():
            pltpu.sync_copy(bm_hbm, bm_v)
            iota = lax.iota(jnp.int32, _B)
            zeros = jnp.zeros((_B,), jnp.int32)

            def p1(t, carry):
                k0, lacc = carry
                bm = bm_v[pl.ds(t * _B, _B)]
                on = bm > 0
                cs = _cumsum16(bm, tmp_v, iota)
                rank = cs - bm + k0
                plsc.store_scatter(perm_v, [jnp.where(on, rank, _TB + iota)],
                                   iota + t * _B)
                gidx2_v[pl.ds(t * _B, _B)] = jnp.where(on, rank, _TB + iota)
                return (k0 + plsc.all_reduce_population_count(on), lacc + bm)

            total, lvec = lax.fori_loop(0, _T, p1, (zeros, zeros))
            len_v[...] = lvec

            def p2(t, k0):
                on = lvec > t
                pmi = jnp.where(on, 1, 0)
                cs = _cumsum16(pmi, tmp_v, iota)
                rank = jnp.minimum(cs - pmi + k0, _TB - 1)
                g = plsc.load_gather(perm_v, [rank])
                gidx_v[pl.ds(t * _B, _B)] = jnp.where(on, g, iota + t * _B)
                return k0 + plsc.all_reduce_population_count(on)

            lax.fori_loop(0, _T, p2, zeros)

            pltpu.sync_copy(gidx_v, gidx_hbm)
            pltpu.sync_copy(gidx2_v, gidx2_hbm)
            pltpu.sync_copy(len_v, len_hbm)

    return idx_kernel


def _build_gather_kernel():
    mesh = _sc_mesh()
    nch = _RPW // _CH

    @functools.partial(
        pl.kernel,
        mesh=mesh,
        compiler_params=pltpu.CompilerParams(needs_layout_passes=False),
        out_type=jax.ShapeDtypeStruct((_TB, _H), jnp.float32),
        scratch_types=(
            [pltpu.VMEM((_RPW,), jnp.int32)]
            + [pltpu.VMEM((_CH, _H), jnp.float32) for _ in range(3)]
            + [pltpu.SemaphoreType.DMA for _ in range(6)]
        ),
    )
    def gather_kernel(table_hbm, idx_hbm, out_hbm, idx_v, *bufs_sems):
        bufs = bufs_sems[:3]
        gsems = bufs_sems[3:6]
        wsems = bufs_sems[6:9]
        wid = lax.axis_index("s") * 2 + lax.axis_index("c")
        base_w = wid * _RPW
        pltpu.sync_copy(idx_hbm.at[pl.ds(base_w, _RPW)], idx_v)

        def gather(c):
            return pltpu.async_copy(
                table_hbm.at[idx_v.at[pl.ds(c * _CH, _CH)]],
                bufs[c % 3], gsems[c % 3])

        def writeback(c):
            return pltpu.async_copy(
                bufs[c % 3], out_hbm.at[pl.ds(base_w + c * _CH, _CH)],
                wsems[c % 3])

        g = {c: gather(c) for c in range(min(3, nch))}
        w = {}
        for c in range(nch):
            g[c].wait()
            w[c] = writeback(c)
            if c + 3 < nch:
                w[c].wait()
                g[c + 3] = gather(c + 3)
        for c in range(max(0, nch - 3), nch):
            w[c].wait()

    return gather_kernel


_TM = 512


def _gi_gemm_body(len_ref, a_ref, w_ref, b_ref, o_ref):
    i = pl.program_id(0)
    maxl = jnp.max(len_ref[...])

    @pl.when(i * _TM < maxl * _B)
    def _():
        acc = lax.dot_general(a_ref[...], w_ref[...],
                              (((1,), (1,)), ((), ())),
                              preferred_element_type=jnp.float32)
        o_ref[...] = acc + b_ref[...]


def _gi_gemm(lcol, compact, w_ih, b_ih_row):
    return pl.pallas_call(
        _gi_gemm_body,
        grid=(_TB // _TM,),
        in_specs=[
            pl.BlockSpec((_B, 1), lambda i: (0, 0)),
            pl.BlockSpec((_TM, _D), lambda i: (i, 0)),
            pl.BlockSpec((3 * _H, _D), lambda i: (0, 0)),
            pl.BlockSpec((1, 3 * _H), lambda i: (0, 0)),
        ],
        out_specs=pl.BlockSpec((_TM, 3 * _H), lambda i: (i, 0)),
        out_shape=jax.ShapeDtypeStruct((_TB, 3 * _H), jnp.float32),
    )(lcol, compact, w_ih, b_ih_row)


_K = 8
_NSC = _T // _K


def _scan_body(len_ref, gi_ref, h0_ref, w_ref, bhh_ref, y_ref, hout_ref, h_v):
    i = pl.program_id(0)
    lvec = len_ref[...]
    maxl = jnp.max(lvec)

    @pl.when(i == 0)
    def _():
        h_v[...] = h0_ref[...]

    @pl.when(jnp.logical_and(i * _K < maxl, i < _NSC))
    def _():
        h = h_v[...]
        for k in range(_K):
            t = i * _K + k
            gi = gi_ref[k]
            gh = lax.dot_general(h.astype(jnp.bfloat16), w_ref[...],
                                 (((1,), (1,)), ((), ())),
                                 preferred_element_type=jnp.float32) + bhh_ref[...]
            r = jax.nn.sigmoid(gi[:, :_H] + gh[:, :_H])
            z = jax.nn.sigmoid(gi[:, _H:2 * _H] + gh[:, _H:2 * _H])
            n = jnp.tanh(gi[:, 2 * _H:] + r * gh[:, 2 * _H:])
            hn = (1.0 - z) * n + z * h
            valid = lvec > t
            h = jnp.where(valid, hn, h)
            y_ref[k] = jnp.where(valid, hn, 0.0)
        h_v[...] = h

    @pl.when(i == _NSC)
    def _():
        y_ref[...] = jnp.zeros((_K, _B, _H), jnp.float32)
        hout_ref[...] = h_v[...]


def _gru_scan(lcol, gi3, h0, w_hh, b_hh_row):
    return pl.pallas_call(
        _scan_body,
        grid=(_NSC + 1,),
        in_specs=[
            pl.BlockSpec((_B, 1), lambda i: (0, 0)),
            pl.BlockSpec((_K, _B, 3 * _H),
                         lambda i: (jnp.minimum(i, _NSC - 1), 0, 0)),
            pl.BlockSpec((_B, _H), lambda i: (0, 0)),
            pl.BlockSpec((3 * _H, _H), lambda i: (0, 0)),
            pl.BlockSpec((1, 3 * _H), lambda i: (0, 0)),
        ],
        out_specs=[
            pl.BlockSpec((_K, _B, _H), lambda i: (i, 0, 0)),
            pl.BlockSpec((_B, _H), lambda i: (0, 0)),
        ],
        out_shape=[
            jax.ShapeDtypeStruct((_T + _K, _B, _H), jnp.float32),
            jax.ShapeDtypeStruct((_B, _H), jnp.float32),
        ],
        scratch_shapes=[pltpu.VMEM((_B, _H), jnp.float32)],
    )(lcol, gi3, h0, w_hh, b_hh_row)


_idx_call = _build_index_kernel()
_gather_rows = _build_gather_kernel()


def kernel(x, rnn_hxs, batch_mask, W_ih, W_hh, b_ih, b_hh):
    x2d = x.reshape(_TB, _D)
    bm = batch_mask.reshape(_TB).astype(jnp.int32)

    gidx, gidx2, lvec = _idx_call(bm)
    lcol = lvec.reshape(_B, 1)

    compact = _gather_rows(x2d, gidx)
    gi = _gi_gemm(lcol, compact, W_ih, b_ih.reshape(1, 3 * _H))
    ypad, h_fin = _gru_scan(lcol, gi.reshape(_T, _B, 3 * _H), rnn_hxs[0],
                            W_hh.astype(jnp.bfloat16), b_hh.reshape(1, 3 * _H))
    ypad2d = ypad.reshape((_T + _K) * _B, _H)
    scores2d = _gather_rows(ypad2d, gidx2)
    return scores2d.reshape(_T, _B, _H), h_fin[None]


# speedup vs baseline: 9.3450x; 1.0326x over previous
import functools

import jax
import jax.numpy as jnp
from jax import lax
from jax.experimental import pallas as pl
from jax.experimental.pallas import tpu as pltpu
from jax.experimental.pallas import tpu_sc as plsc

_T, _B, _D, _H = 512, 16, 512, 512
_TB = _T * _B
_NW = 32
_RPW = _TB // _NW
_CH = 64


def _sc_mesh():
    return plsc.VectorSubcoreMesh(core_axis_name="c", subcore_axis_name="s")


def _cumsum16(v, tmp_v, iota):
    for s in (1, 2, 4, 8):
        tmp_v[...] = v
        sh = plsc.load_gather(tmp_v, [jnp.maximum(iota - s, 0)])
        v = v + jnp.where(iota >= s, sh, 0)
    return v


def _build_index_kernel():
    mesh = _sc_mesh()

    @functools.partial(
        pl.kernel,
        mesh=mesh,
        compiler_params=pltpu.CompilerParams(needs_layout_passes=False),
        out_type=[
            jax.ShapeDtypeStruct((_TB,), jnp.int32),
            jax.ShapeDtypeStruct((_TB,), jnp.int32),
            jax.ShapeDtypeStruct((_B,), jnp.int32),
        ],
        scratch_types=[
            pltpu.VMEM((_TB,), jnp.int32),
            pltpu.VMEM((_TB + _B,), jnp.int32),
            pltpu.VMEM((_TB,), jnp.int32),
            pltpu.VMEM((_TB,), jnp.int32),
            pltpu.VMEM((_B,), jnp.int32),
            pltpu.VMEM((_B,), jnp.int32),
        ],
    )
    def idx_kernel(bm_hbm, gidx_hbm, gidx2_hbm, len_hbm,
                   bm_v, perm_v, gidx_v, gidx2_v, len_v, tmp_v):
        wid = lax.axis_index("s") * 2 + lax.axis_index("c")

        @pl.when(wid == 0)
        def _():
            pltpu.sync_copy(bm_hbm, bm_v)
            iota = lax.iota(jnp.int32, _B)
            zeros = jnp.zeros((_B,), jnp.int32)

            def p1(t, carry):
                k0, lacc = carry
                bm = bm_v[pl.ds(t * _B, _B)]
                on = bm > 0
                cs = _cumsum16(bm, tmp_v, iota)
                rank = cs - bm + k0
                plsc.store_scatter(perm_v, [jnp.where(on, rank, _TB + iota)],
                                   iota + t * _B)
                gidx2_v[pl.ds(t * _B, _B)] = jnp.where(on, rank, _TB + iota)
                return (k0 + plsc.all_reduce_population_count(on), lacc + bm)

            total, lvec = lax.fori_loop(0, _T, p1, (zeros, zeros))
            len_v[...] = lvec

            def p2(t, k0):
                on = lvec > t
                pmi = jnp.where(on, 1, 0)
                cs = _cumsum16(pmi, tmp_v, iota)
                rank = jnp.minimum(cs - pmi + k0, _TB - 1)
                g = plsc.load_gather(perm_v, [rank])
                gidx_v[pl.ds(t * _B, _B)] = jnp.where(on, g, iota + t * _B)
                return k0 + plsc.all_reduce_population_count(on)

            lax.fori_loop(0, _T, p2, zeros)

            pltpu.sync_copy(gidx_v, gidx_hbm)
            pltpu.sync_copy(gidx2_v, gidx2_hbm)
            pltpu.sync_copy(len_v, len_hbm)

    return idx_kernel


def _build_gather_kernel():
    mesh = _sc_mesh()
    nch = _RPW // _CH

    @functools.partial(
        pl.kernel,
        mesh=mesh,
        compiler_params=pltpu.CompilerParams(needs_layout_passes=False),
        out_type=jax.ShapeDtypeStruct((_TB, _H), jnp.float32),
        scratch_types=(
            [pltpu.VMEM((_RPW,), jnp.int32)]
            + [pltpu.VMEM((_CH, _H), jnp.float32) for _ in range(3)]
            + [pltpu.SemaphoreType.DMA for _ in range(6)]
        ),
    )
    def gather_kernel(table_hbm, idx_hbm, out_hbm, idx_v, *bufs_sems):
        bufs = bufs_sems[:3]
        gsems = bufs_sems[3:6]
        wsems = bufs_sems[6:9]
        wid = lax.axis_index("s") * 2 + lax.axis_index("c")
        base_w = wid * _RPW
        pltpu.sync_copy(idx_hbm.at[pl.ds(base_w, _RPW)], idx_v)

        def gather(c):
            return pltpu.async_copy(
                table_hbm.at[idx_v.at[pl.ds(c * _CH, _CH)]],
                bufs[c % 3], gsems[c % 3])

        def writeback(c):
            return pltpu.async_copy(
                bufs[c % 3], out_hbm.at[pl.ds(base_w + c * _CH, _CH)],
                wsems[c % 3])

        g = {c: gather(c) for c in range(min(3, nch))}
        w = {}
        for c in range(nch):
            g[c].wait()
            w[c] = writeback(c)
            if c + 3 < nch:
                w[c].wait()
                g[c + 3] = gather(c + 3)
        for c in range(max(0, nch - 3), nch):
            w[c].wait()

    return gather_kernel


_TM = 512


def _gi_gemm_body(len_ref, a_ref, w_ref, b_ref, o_ref):
    i = pl.program_id(0)
    maxl = jnp.max(len_ref[...])

    @pl.when(i * _TM < maxl * _B)
    def _():
        acc = lax.dot_general(a_ref[...], w_ref[...],
                              (((1,), (1,)), ((), ())),
                              preferred_element_type=jnp.float32)
        o_ref[...] = acc + b_ref[...]


def _gi_gemm(lcol, compact, w_ih, b_ih_row):
    return pl.pallas_call(
        _gi_gemm_body,
        grid=(_TB // _TM,),
        in_specs=[
            pl.BlockSpec((_B, 1), lambda i: (0, 0)),
            pl.BlockSpec((_TM, _D), lambda i: (i, 0)),
            pl.BlockSpec((3 * _H, _D), lambda i: (0, 0)),
            pl.BlockSpec((1, 3 * _H), lambda i: (0, 0)),
        ],
        out_specs=pl.BlockSpec((_TM, 3 * _H), lambda i: (i, 0)),
        out_shape=jax.ShapeDtypeStruct((_TB, 3 * _H), jnp.float32),
    )(lcol, compact, w_ih, b_ih_row)


_K = 16
_NSC = _T // _K


def _scan_body(len_ref, gi_ref, h0_ref, w_ref, bhh_ref, y_ref, hout_ref, h_v):
    i = pl.program_id(0)
    lvec = len_ref[...]
    maxl = jnp.max(lvec)

    @pl.when(i == 0)
    def _():
        h_v[...] = h0_ref[...]

    @pl.when(jnp.logical_and(i * _K < maxl, i < _NSC))
    def _():
        h = h_v[...]
        for k in range(_K):
            t = i * _K + k
            gi = gi_ref[k]
            gh = lax.dot_general(h.astype(jnp.bfloat16), w_ref[...],
                                 (((1,), (1,)), ((), ())),
                                 preferred_element_type=jnp.float32) + bhh_ref[...]
            r = jax.nn.sigmoid(gi[:, :_H] + gh[:, :_H])
            z = jax.nn.sigmoid(gi[:, _H:2 * _H] + gh[:, _H:2 * _H])
            n = jnp.tanh(gi[:, 2 * _H:] + r * gh[:, 2 * _H:])
            hn = (1.0 - z) * n + z * h
            valid = lvec > t
            h = jnp.where(valid, hn, h)
            y_ref[k] = jnp.where(valid, hn, 0.0)
        h_v[...] = h

    @pl.when(i == _NSC)
    def _():
        y_ref[...] = jnp.zeros((_K, _B, _H), jnp.float32)
        hout_ref[...] = h_v[...]


def _gru_scan(lcol, gi3, h0, w_hh, b_hh_row):
    return pl.pallas_call(
        _scan_body,
        grid=(_NSC + 1,),
        in_specs=[
            pl.BlockSpec((_B, 1), lambda i: (0, 0)),
            pl.BlockSpec((_K, _B, 3 * _H),
                         lambda i: (jnp.minimum(i, _NSC - 1), 0, 0)),
            pl.BlockSpec((_B, _H), lambda i: (0, 0)),
            pl.BlockSpec((3 * _H, _H), lambda i: (0, 0)),
            pl.BlockSpec((1, 3 * _H), lambda i: (0, 0)),
        ],
        out_specs=[
            pl.BlockSpec((_K, _B, _H), lambda i: (i, 0, 0)),
            pl.BlockSpec((_B, _H), lambda i: (0, 0)),
        ],
        out_shape=[
            jax.ShapeDtypeStruct((_T + _K, _B, _H), jnp.float32),
            jax.ShapeDtypeStruct((_B, _H), jnp.float32),
        ],
        scratch_shapes=[pltpu.VMEM((_B, _H), jnp.float32)],
    )(lcol, gi3, h0, w_hh, b_hh_row)


_idx_call = _build_index_kernel()
_gather_rows = _build_gather_kernel()


def kernel(x, rnn_hxs, batch_mask, W_ih, W_hh, b_ih, b_hh):
    x2d = x.reshape(_TB, _D)
    bm = batch_mask.reshape(_TB).astype(jnp.int32)

    gidx, gidx2, lvec = _idx_call(bm)
    lcol = lvec.reshape(_B, 1)

    compact = _gather_rows(x2d, gidx)
    gi = _gi_gemm(lcol, compact, W_ih, b_ih.reshape(1, 3 * _H))
    ypad, h_fin = _gru_scan(lcol, gi.reshape(_T, _B, 3 * _H), rnn_hxs[0],
                            W_hh.astype(jnp.bfloat16), b_hh.reshape(1, 3 * _H))
    ypad2d = ypad.reshape((_T + _K) * _B, _H)
    scores2d = _gather_rows(ypad2d, gidx2)
    return scores2d.reshape(_T, _B, _H), h_fin[None]


# speedup vs baseline: 9.5738x; 1.0245x over previous
import functools

import jax
import jax.numpy as jnp
from jax import lax
from jax.experimental import pallas as pl
from jax.experimental.pallas import tpu as pltpu
from jax.experimental.pallas import tpu_sc as plsc

_T, _B, _D, _H = 512, 16, 512, 512
_TB = _T * _B
_NW = 32
_RPW = _TB // _NW
_CH = 64


def _sc_mesh():
    return plsc.VectorSubcoreMesh(core_axis_name="c", subcore_axis_name="s")


def _cumsum16(v, tmp_v, iota):
    for s in (1, 2, 4, 8):
        tmp_v[...] = v
        sh = plsc.load_gather(tmp_v, [jnp.maximum(iota - s, 0)])
        v = v + jnp.where(iota >= s, sh, 0)
    return v


def _build_index_kernel():
    mesh = _sc_mesh()

    @functools.partial(
        pl.kernel,
        mesh=mesh,
        compiler_params=pltpu.CompilerParams(needs_layout_passes=False),
        out_type=[
            jax.ShapeDtypeStruct((_TB,), jnp.int32),
            jax.ShapeDtypeStruct((_TB,), jnp.int32),
            jax.ShapeDtypeStruct((_B,), jnp.int32),
        ],
        scratch_types=[
            pltpu.VMEM((_TB,), jnp.int32),
            pltpu.VMEM((_TB + _B,), jnp.int32),
            pltpu.VMEM((_TB,), jnp.int32),
            pltpu.VMEM((_TB,), jnp.int32),
            pltpu.VMEM((_B,), jnp.int32),
            pltpu.VMEM((_B,), jnp.int32),
        ],
    )
    def idx_kernel(bm_hbm, gidx_hbm, gidx2_hbm, len_hbm,
                   bm_v, perm_v, gidx_v, gidx2_v, len_v, tmp_v):
        wid = lax.axis_index("s") * 2 + lax.axis_index("c")

        @pl.when(wid == 0)
        def _():
            pltpu.sync_copy(bm_hbm, bm_v)
            iota = lax.iota(jnp.int32, _B)
            zeros = jnp.zeros((_B,), jnp.int32)

            def p1(t, carry):
                k0, lacc = carry
                bm = bm_v[pl.ds(t * _B, _B)]
                on = bm > 0
                cs = _cumsum16(bm, tmp_v, iota)
                rank = cs - bm + k0
                plsc.store_scatter(perm_v, [jnp.where(on, rank, _TB + iota)],
                                   iota + t * _B)
                gidx2_v[pl.ds(t * _B, _B)] = jnp.where(on, rank, _TB + iota)
                return (k0 + plsc.all_reduce_population_count(on), lacc + bm)

            total, lvec = lax.fori_loop(0, _T, p1, (zeros, zeros))
            len_v[...] = lvec

            def p2(t, k0):
                on = lvec > t
                pmi = jnp.where(on, 1, 0)
                cs = _cumsum16(pmi, tmp_v, iota)
                rank = jnp.minimum(cs - pmi + k0, _TB - 1)
                g = plsc.load_gather(perm_v, [rank])
                gidx_v[pl.ds(t * _B, _B)] = jnp.where(on, g, iota + t * _B)
                return k0 + plsc.all_reduce_population_count(on)

            lax.fori_loop(0, _T, p2, zeros)

            pltpu.sync_copy(gidx_v, gidx_hbm)
            pltpu.sync_copy(gidx2_v, gidx2_hbm)
            pltpu.sync_copy(len_v, len_hbm)

    return idx_kernel


def _build_gather_kernel():
    mesh = _sc_mesh()
    nch = _RPW // _CH

    @functools.partial(
        pl.kernel,
        mesh=mesh,
        compiler_params=pltpu.CompilerParams(needs_layout_passes=False),
        out_type=jax.ShapeDtypeStruct((_TB, _H), jnp.float32),
        scratch_types=(
            [pltpu.VMEM((_RPW,), jnp.int32)]
            + [pltpu.VMEM((_CH, _H), jnp.float32) for _ in range(3)]
            + [pltpu.SemaphoreType.DMA for _ in range(6)]
        ),
    )
    def gather_kernel(table_hbm, idx_hbm, out_hbm, idx_v, *bufs_sems):
        bufs = bufs_sems[:3]
        gsems = bufs_sems[3:6]
        wsems = bufs_sems[6:9]
        wid = lax.axis_index("s") * 2 + lax.axis_index("c")
        base_w = wid * _RPW
        pltpu.sync_copy(idx_hbm.at[pl.ds(base_w, _RPW)], idx_v)

        def gather(c):
            return pltpu.async_copy(
                table_hbm.at[idx_v.at[pl.ds(c * _CH, _CH)]],
                bufs[c % 3], gsems[c % 3])

        def writeback(c):
            return pltpu.async_copy(
                bufs[c % 3], out_hbm.at[pl.ds(base_w + c * _CH, _CH)],
                wsems[c % 3])

        g = {c: gather(c) for c in range(min(3, nch))}
        w = {}
        for c in range(nch):
            g[c].wait()
            w[c] = writeback(c)
            if c + 3 < nch:
                w[c].wait()
                g[c + 3] = gather(c + 3)
        for c in range(max(0, nch - 3), nch):
            w[c].wait()

    return gather_kernel


_TM = 512


def _gi_gemm_body(len_ref, a_ref, w_ref, b_ref, o_ref):
    i = pl.program_id(0)
    maxl = jnp.max(len_ref[...])

    @pl.when(i * _TM < maxl * _B)
    def _():
        acc = lax.dot_general(a_ref[...], w_ref[...],
                              (((1,), (1,)), ((), ())),
                              preferred_element_type=jnp.float32)
        o_ref[...] = acc + b_ref[...]


def _gi_gemm(lcol, compact, w_ih, b_ih_row):
    return pl.pallas_call(
        _gi_gemm_body,
        grid=(_TB // _TM,),
        in_specs=[
            pl.BlockSpec((_B, 1), lambda i: (0, 0)),
            pl.BlockSpec((_TM, _D), lambda i: (i, 0)),
            pl.BlockSpec((3 * _H, _D), lambda i: (0, 0)),
            pl.BlockSpec((1, 3 * _H), lambda i: (0, 0)),
        ],
        out_specs=pl.BlockSpec((_TM, 3 * _H), lambda i: (i, 0)),
        out_shape=jax.ShapeDtypeStruct((_TB, 3 * _H), jnp.float32),
    )(lcol, compact, w_ih, b_ih_row)


_K = 32
_NSC = _T // _K


def _scan_body(len_ref, gi_ref, h0_ref, w_ref, bhh_ref, y_ref, hout_ref, h_v):
    i = pl.program_id(0)
    lvec = len_ref[...]
    maxl = jnp.max(lvec)

    @pl.when(i == 0)
    def _():
        h_v[...] = h0_ref[...]

    @pl.when(jnp.logical_and(i * _K < maxl, i < _NSC))
    def _():
        h = h_v[...]
        for k in range(_K):
            t = i * _K + k
            gi = gi_ref[k]
            gh = lax.dot_general(h.astype(jnp.bfloat16), w_ref[...],
                                 (((1,), (1,)), ((), ())),
                                 preferred_element_type=jnp.float32) + bhh_ref[...]
            r = jax.nn.sigmoid(gi[:, :_H] + gh[:, :_H])
            z = jax.nn.sigmoid(gi[:, _H:2 * _H] + gh[:, _H:2 * _H])
            n = jnp.tanh(gi[:, 2 * _H:] + r * gh[:, 2 * _H:])
            hn = (1.0 - z) * n + z * h
            valid = lvec > t
            h = jnp.where(valid, hn, h)
            y_ref[k] = jnp.where(valid, hn, 0.0)
        h_v[...] = h

    @pl.when(i == _NSC)
    def _():
        y_ref[...] = jnp.zeros((_K, _B, _H), jnp.float32)
        hout_ref[...] = h_v[...]


def _gru_scan(lcol, gi3, h0, w_hh, b_hh_row):
    return pl.pallas_call(
        _scan_body,
        grid=(_NSC + 1,),
        in_specs=[
            pl.BlockSpec((_B, 1), lambda i: (0, 0)),
            pl.BlockSpec((_K, _B, 3 * _H),
                         lambda i: (jnp.minimum(i, _NSC - 1), 0, 0)),
            pl.BlockSpec((_B, _H), lambda i: (0, 0)),
            pl.BlockSpec((3 * _H, _H), lambda i: (0, 0)),
            pl.BlockSpec((1, 3 * _H), lambda i: (0, 0)),
        ],
        out_specs=[
            pl.BlockSpec((_K, _B, _H), lambda i: (i, 0, 0)),
            pl.BlockSpec((_B, _H), lambda i: (0, 0)),
        ],
        out_shape=[
            jax.ShapeDtypeStruct((_T + _K, _B, _H), jnp.float32),
            jax.ShapeDtypeStruct((_B, _H), jnp.float32),
        ],
        scratch_shapes=[pltpu.VMEM((_B, _H), jnp.float32)],
    )(lcol, gi3, h0, w_hh, b_hh_row)


_idx_call = _build_index_kernel()
_gather_rows = _build_gather_kernel()


def kernel(x, rnn_hxs, batch_mask, W_ih, W_hh, b_ih, b_hh):
    x2d = x.reshape(_TB, _D)
    bm = batch_mask.reshape(_TB).astype(jnp.int32)

    gidx, gidx2, lvec = _idx_call(bm)
    lcol = lvec.reshape(_B, 1)

    compact = _gather_rows(x2d, gidx)
    gi = _gi_gemm(lcol, compact, W_ih, b_ih.reshape(1, 3 * _H))
    ypad, h_fin = _gru_scan(lcol, gi.reshape(_T, _B, 3 * _H), rnn_hxs[0],
                            W_hh.astype(jnp.bfloat16), b_hh.reshape(1, 3 * _H))
    ypad2d = ypad.reshape((_T + _K) * _B, _H)
    scores2d = _gather_rows(ypad2d, gidx2)
    return scores2d.reshape(_T, _B, _H), h_fin[None]
